# baseline (speedup 1.0000x reference)
import jax
import jax.numpy as jnp
from jax.experimental import pallas as pl
from jax.experimental.pallas import tpu as pltpu

B, N, C, IN, HID, ATOMS = 2, 4096, 1024, 128, 64, 64
BLK = 512
NBLK = N // BLK
SCALE = HID ** -0.5
W2 = 2 * HID
NC1 = 16

R_NMW, R_CMW, R_QW, R_KW, R_AT = 0, 128, 256, 320, 384
R_SW = 448
R_C1W = 512
R_C2W = 576
R_FW = 584
R_BIAS = 712
(B_NM, B_CM, B_Q, B_K, B_S, B_C1, B_C2, B_PCG, B_PCB, B_F, B_NG,
 B_NB) = [R_BIAS + 8 * k for k in range(12)]
PROWS = R_BIAS + 8 * 12


def _dot(a, b):
    return jnp.dot(a, b, preferred_element_type=jnp.float32)


def _mmt(a, w):
    return jax.lax.dot_general(a, w, (((1,), (1,)), ((), ())),
                               preferred_element_type=jnp.float32)


def _ln(x, g, b, eps=1e-5):
    m = jnp.mean(x, axis=1, keepdims=True)
    xc = x - m
    v = jnp.mean(xc * xc, axis=1, keepdims=True)
    return xc / jnp.sqrt(v + eps) * g + b


def _fused(x_ref, adj_ref, inc_ref, gate_ref, pp,
           out_ref, ent_ref, inc_s, h_s, cell_s, p_s):
    p = pl.program_id(0)
    i = pl.program_id(1)

    @pl.when(p == 0)
    def _phase0():
        @pl.when(i == 0)
        def _():
            cell_s[...] = jnp.zeros_like(cell_s)

        gate = gate_ref[...]
        inc_blk = inc_ref[...]
        inc_s[pl.ds(i * BLK, BLK), :] = inc_blk
        for b in range(B):
            xg = x_ref[b] * gate
            h_s[pl.ds(i * BLK, BLK), b * HID:(b + 1) * HID] = (
                _dot(xg, pp[R_NMW:R_NMW + IN, :HID]) + pp[B_NM:B_NM + 1, :HID])
            cell_s[:, b * IN:(b + 1) * IN] += jax.lax.dot_general(
                inc_blk, xg, (((0,), (0,)), ((), ())),
                preferred_element_type=jnp.float32)

    @pl.when(p == 1)
    def _phase1():
        @pl.when(i == 0)
        def _cell_stage():
            kk = (_dot(pp[R_AT:R_AT + ATOMS, :HID],
                       pp[R_KW:R_KW + HID, :HID]) + pp[B_K:B_K + 1, :HID])
            ent = jnp.float32(0.0)
            for b in range(B):
                cell_b = cell_s[:, b * IN:(b + 1) * IN]
                h2 = (_dot(cell_b, pp[R_CMW:R_CMW + IN, :HID])
                      + pp[B_CM:B_CM + 1, :HID])
                q = (_dot(h2, pp[R_QW:R_QW + HID, :HID])
                     + pp[B_Q:B_Q + 1, :HID])
                attn = _mmt(q, kk) * SCALE
                m = jnp.max(attn, axis=1, keepdims=True)
                e = jnp.exp(attn - m)
                w = e / jnp.sum(e, axis=1, keepdims=True)
                p_s[:, b * HID:(b + 1) * HID] = _dot(
                    w, pp[R_AT:R_AT + ATOMS, :HID])
                ent = ent - jnp.sum(w * jnp.log(w + 1e-6))
            ent_ref[...] = jnp.reshape(ent / (B * C), (1, 1))

        agg = _dot(adj_ref[...], h_s[...])
        pn = _dot(inc_s[pl.ds(i * BLK, BLK), :], p_s[...])
        for b in range(B):
            ha = agg[:, b * HID:(b + 1) * HID]
            pnb = pn[:, b * HID:(b + 1) * HID]
            sur = ha - pnb
            err = jnp.sqrt(jnp.sum(sur * sur, axis=1, keepdims=True))
            conf = 1.0 / (1.0 + err)
            ps = (_dot(sur, pp[R_SW:R_SW + HID, :HID])
                  + pp[B_S:B_S + 1, :HID])
            r = jnp.maximum(
                _dot(jnp.abs(sur), pp[R_C1W:R_C1W + HID, :NC1])
                + pp[B_C1:B_C1 + 1, :NC1], 0.0)
            lc = jax.nn.sigmoid(
                jnp.sum(r * pp[R_C2W:R_C2W + 1, :NC1], axis=1, keepdims=True)
                + pp[B_C2:B_C2 + 1, :1])
            ge = ps * (conf * lc)
            processed = _ln(ge + ha, pp[B_PCG:B_PCG + 1, :HID],
                            pp[B_PCB:B_PCB + 1, :HID])
            comb = jnp.concatenate([processed, pnb], axis=1)
            o = _dot(comb, pp[R_FW:R_FW + W2, :HID]) + pp[B_F:B_F + 1, :HID]
            out_ref[b] = _ln(o, pp[B_NG:B_NG + 1, :HID],
                             pp[B_NB:B_NB + 1, :HID])


def kernel(x_nodes, adjacency, incidence, node_importance, nm_w, nm_b, cm_w,
           cm_b, atoms, q_w, q_b, k_w, k_b, s_w, s_b, c1_w, c1_b, c2_w, c2_b,
           pc_g, pc_b, f_w, f_b, n_g, n_b):
    f32 = jnp.float32
    gate2d = jnp.broadcast_to(
        jax.nn.sigmoid(node_importance)[:, None], (N, IN))

    pp = jnp.zeros((PROWS, W2), f32)
    pp = (pp
          .at[R_NMW:R_NMW + IN, :HID].set(nm_w.T)
          .at[R_CMW:R_CMW + IN, :HID].set(cm_w.T)
          .at[R_QW:R_QW + HID, :HID].set(q_w.T)
          .at[R_KW:R_KW + HID, :HID].set(k_w.T)
          .at[R_AT:R_AT + ATOMS, :HID].set(atoms)
          .at[R_SW:R_SW + HID, :HID].set(s_w.T)
          .at[R_C1W:R_C1W + HID, :NC1].set(c1_w.T)
          .at[R_C2W, :NC1].set(c2_w[0])
          .at[R_FW:R_FW + W2, :HID].set(f_w.T)
          .at[B_NM, :HID].set(nm_b)
          .at[B_CM, :HID].set(cm_b)
          .at[B_Q, :HID].set(q_b)
          .at[B_K, :HID].set(k_b)
          .at[B_S, :HID].set(s_b)
          .at[B_C1, :NC1].set(c1_b)
          .at[B_C2, 0].set(c2_b[0])
          .at[B_PCG, :HID].set(pc_g)
          .at[B_PCB, :HID].set(pc_b)
          .at[B_F, :HID].set(f_b)
          .at[B_NG, :HID].set(n_g)
          .at[B_NB, :HID].set(n_b))

    last = NBLK - 1
    in_specs = [
        pl.BlockSpec((B, BLK, IN), lambda p, i: (0, (1 - p) * i + p * last, 0)),
        pl.BlockSpec((BLK, N), lambda p, i: (p * i, 0)),
        pl.BlockSpec((BLK, C), lambda p, i: ((1 - p) * i + p * last, 0)),
        pl.BlockSpec((BLK, IN), lambda p, i: ((1 - p) * i + p * last, 0)),
        pl.BlockSpec(pp.shape, lambda p, i: (0, 0)),
    ]

    out, ent = pl.pallas_call(
        _fused,
        grid=(2, NBLK),
        in_specs=in_specs,
        out_specs=[
            pl.BlockSpec((B, BLK, HID), lambda p, i: (0, p * i, 0)),
            pl.BlockSpec((1, 1), lambda p, i: (0, 0)),
        ],
        out_shape=[
            jax.ShapeDtypeStruct((B, N, HID), f32),
            jax.ShapeDtypeStruct((1, 1), f32),
        ],
        scratch_shapes=[
            pltpu.VMEM((N, C), f32),
            pltpu.VMEM((N, W2), f32),
            pltpu.VMEM((C, B * IN), f32),
            pltpu.VMEM((C, W2), f32),
        ],
        compiler_params=pltpu.CompilerParams(
            dimension_semantics=("arbitrary", "arbitrary")),
    )(x_nodes, adjacency, incidence, gate2d, pp)
    return out, ent[0, 0]


# speedup vs baseline: 1.3532x; 1.3532x over previous
import jax
import jax.numpy as jnp
from jax.experimental import pallas as pl
from jax.experimental.pallas import tpu as pltpu

B, N, C, IN, HID, ATOMS = 2, 4096, 1024, 128, 64, 64
BLK = 512
NBLK = N // BLK
SCALE = HID ** -0.5


def _mmt(a, w):
    return jax.lax.dot_general(a, w, (((1,), (1,)), ((), ())),
                               preferred_element_type=jnp.float32)


def _ln(x, g, b, eps=1e-5):
    m = jnp.mean(x, axis=1, keepdims=True)
    xc = x - m
    v = jnp.mean(xc * xc, axis=1, keepdims=True)
    return xc / jnp.sqrt(v + eps) * g + b


def _fused(x_ref, adj_ref, inc_ref, imp_ref,
           nm_w, nm_b, cm_w, cm_b, atoms,
           q_w, q_b, k_w, k_b, s_w, s_b,
           c1_w, c1_b, c2_w, c2_b,
           pc_g, pc_b, f_w, f_b, n_g, n_b,
           out_ref, ent_ref,
           inc_s, h_s, cell_s, p_s):
    p = pl.program_id(0)
    i = pl.program_id(1)

    @pl.when(p == 0)
    def _phase0():
        @pl.when(i == 0)
        def _():
            cell_s[...] = jnp.zeros_like(cell_s)

        gate = jax.nn.sigmoid(imp_ref[...])
        inc_blk = inc_ref[...]
        inc_s[pl.ds(i * BLK, BLK), :] = inc_blk
        hs, cs = [], []
        for b in range(B):
            xg = x_ref[b] * gate
            hs.append(_mmt(xg, nm_w[...]) + nm_b[...])
            cs.append(jax.lax.dot_general(
                inc_blk, xg, (((0,), (0,)), ((), ())),
                preferred_element_type=jnp.float32))
        h_s[pl.ds(i * BLK, BLK), :] = jnp.concatenate(hs, axis=1)
        cell_s[...] += jnp.concatenate(cs, axis=1)

    @pl.when(p == 1)
    def _phase1():
        @pl.when(i == 0)
        def _cell_stage():
            kk = _mmt(atoms[...], k_w[...]) + k_b[...]
            ent = jnp.float32(0.0)
            for b in range(B):
                cell_b = cell_s[:, b * IN:(b + 1) * IN]
                h2 = _mmt(cell_b, cm_w[...]) + cm_b[...]
                q = _mmt(h2, q_w[...]) + q_b[...]
                attn = _mmt(q, kk) * SCALE
                m = jnp.max(attn, axis=1, keepdims=True)
                e = jnp.exp(attn - m)
                w = e / jnp.sum(e, axis=1, keepdims=True)
                pc = jnp.dot(w, atoms[...],
                             preferred_element_type=jnp.float32)
                p_s[:, b * HID:(b + 1) * HID] = pc
                ent = ent - jnp.sum(w * jnp.log(w + 1e-6))
            ent_ref[...] = jnp.reshape(ent / (B * C), (1, 1))

        agg = jnp.dot(adj_ref[...], h_s[...],
                      preferred_element_type=jnp.float32)
        pn = jnp.dot(inc_s[pl.ds(i * BLK, BLK), :], p_s[...],
                     preferred_element_type=jnp.float32)
        for b in range(B):
            ha = agg[:, b * HID:(b + 1) * HID]
            pnb = pn[:, b * HID:(b + 1) * HID]
            sur = ha - pnb
            err = jnp.sqrt(jnp.sum(sur * sur, axis=1, keepdims=True))
            conf = 1.0 / (1.0 + err)
            ps = _mmt(sur, s_w[...]) + s_b[...]
            r = jnp.maximum(_mmt(jnp.abs(sur), c1_w[...]) + c1_b[...], 0.0)
            lc = jax.nn.sigmoid(
                jnp.sum(r * c2_w[...], axis=1, keepdims=True) + c2_b[...])
            ge = ps * (conf * lc)
            processed = _ln(ge + ha, pc_g[...], pc_b[...])
            comb = jnp.concatenate([processed, pnb], axis=1)
            o = _mmt(comb, f_w[...]) + f_b[...]
            out_ref[b] = _ln(o, n_g[...], n_b[...])


def kernel(x_nodes, adjacency, incidence, node_importance, nm_w, nm_b, cm_w,
           cm_b, atoms, q_w, q_b, k_w, k_b, s_w, s_b, c1_w, c1_b, c2_w, c2_b,
           pc_g, pc_b, f_w, f_b, n_g, n_b):
    f32 = jnp.float32
    row = lambda v: jnp.reshape(v, (1, -1))
    imp = jnp.reshape(node_importance, (N, 1))

    def full(a):
        return pl.BlockSpec(a.shape, lambda p, i: (0,) * a.ndim)

    last = NBLK - 1
    in_specs = [
        pl.BlockSpec((B, BLK, IN), lambda p, i: (0, (1 - p) * i + p * last, 0)),
        pl.BlockSpec((BLK, N), lambda p, i: (p * i, 0)),
        pl.BlockSpec((BLK, C), lambda p, i: ((1 - p) * i + p * last, 0)),
        pl.BlockSpec((BLK, 1), lambda p, i: ((1 - p) * i + p * last, 0)),
    ]
    smalls = [nm_w, row(nm_b), cm_w, row(cm_b), atoms,
              q_w, row(q_b), k_w, row(k_b), s_w, row(s_b),
              c1_w, row(c1_b), c2_w, row(c2_b),
              row(pc_g), row(pc_b), f_w, row(f_b), row(n_g), row(n_b)]
    in_specs += [full(a) for a in smalls]

    out, ent = pl.pallas_call(
        _fused,
        grid=(2, NBLK),
        in_specs=in_specs,
        out_specs=[
            pl.BlockSpec((B, BLK, HID), lambda p, i: (0, p * i, 0)),
            pl.BlockSpec((1, 1), lambda p, i: (0, 0)),
        ],
        out_shape=[
            jax.ShapeDtypeStruct((B, N, HID), f32),
            jax.ShapeDtypeStruct((1, 1), f32),
        ],
        scratch_shapes=[
            pltpu.VMEM((N, C), f32),
            pltpu.VMEM((N, B * HID), f32),
            pltpu.VMEM((C, B * IN), f32),
            pltpu.VMEM((C, B * HID), f32),
        ],
        compiler_params=pltpu.CompilerParams(
            dimension_semantics=("arbitrary", "arbitrary")),
    )(x_nodes, adjacency, incidence, imp, *smalls)
    return out, ent[0, 0]
